# Initial kernel scaffold; baseline (speedup 1.0000x reference)
import jax
import jax.numpy as jnp
from jax.experimental import pallas as pl


def kernel(x):
    raise NotImplementedError("write your pallas kernel here")


# speedup vs baseline: 1.5478x; 1.5478x over previous
import jax
import jax.numpy as jnp
from jax.experimental import pallas as pl
from jax.experimental.pallas import tpu as pltpu

B, S, L = 4, 8192, 4096
SBLK = 512
NS = S // SBLK


def _argmin_body(x_ref, o_ref, mv_ref, mi_ref):
    s = pl.program_id(1)
    v = x_ref[0]
    m = jnp.min(v, axis=0)
    iota = jax.lax.broadcasted_iota(jnp.int32, v.shape, 0)
    idx = jnp.min(jnp.where(v == m[None, :], iota, jnp.int32(S)), axis=0) + s * SBLK

    @pl.when(s == 0)
    def _():
        mv_ref[0] = m
        mi_ref[0] = idx

    @pl.when(s > 0)
    def _():
        better = m < mv_ref[0]
        mi_ref[0] = jnp.where(better, idx, mi_ref[0])
        mv_ref[0] = jnp.where(better, m, mv_ref[0])

    @pl.when(s == NS - 1)
    def _():
        o_ref[0] = mi_ref[...]


def kernel(x):
    out = pl.pallas_call(
        _argmin_body,
        grid=(B, NS),
        in_specs=[pl.BlockSpec((1, SBLK, L), lambda b, s: (b, s, 0))],
        out_specs=pl.BlockSpec((1, 1, L), lambda b, s: (b, 0, 0)),
        out_shape=jax.ShapeDtypeStruct((B, 1, L), jnp.int32),
        scratch_shapes=[
            pltpu.VMEM((1, L), jnp.float32),
            pltpu.VMEM((1, L), jnp.int32),
        ],
    )(x)
    return out.reshape(B, L)


# speedup vs baseline: 1.7223x; 1.1128x over previous
import jax
import jax.numpy as jnp
from jax.experimental import pallas as pl
from jax.experimental.pallas import tpu as pltpu

B, S, L = 4, 8192, 4096
SBLK = 1024
NS = S // SBLK


def _argmin_body(x_ref, o_ref, mv_ref, mi_ref):
    s = pl.program_id(1)
    v = x_ref[0]
    m = jnp.min(v, axis=0)
    iota = jax.lax.broadcasted_iota(jnp.int32, v.shape, 0)
    idx = jnp.min(jnp.where(v == m[None, :], iota, jnp.int32(S)), axis=0) + s * SBLK

    @pl.when(s == 0)
    def _():
        mv_ref[0] = m
        mi_ref[0] = idx

    @pl.when(s > 0)
    def _():
        better = m < mv_ref[0]
        mi_ref[0] = jnp.where(better, idx, mi_ref[0])
        mv_ref[0] = jnp.where(better, m, mv_ref[0])

    @pl.when(s == NS - 1)
    def _():
        o_ref[0] = mi_ref[...]


def kernel(x):
    out = pl.pallas_call(
        _argmin_body,
        grid=(B, NS),
        in_specs=[pl.BlockSpec((1, SBLK, L), lambda b, s: (b, s, 0))],
        out_specs=pl.BlockSpec((1, 1, L), lambda b, s: (b, 0, 0)),
        out_shape=jax.ShapeDtypeStruct((B, 1, L), jnp.int32),
        scratch_shapes=[
            pltpu.VMEM((1, L), jnp.float32),
            pltpu.VMEM((1, L), jnp.int32),
        ],
    )(x)
    return out.reshape(B, L)
